# baseline (speedup 1.0000x reference)
import functools

import jax
import jax.numpy as jnp
from jax.experimental import pallas as pl
from jax.experimental.pallas import tpu as pltpu

Q, M, D, U, C, K = 1024, 100000, 512, 256, 1000, 64
BN = 2048
M_PAD = ((M + BN - 1) // BN) * BN


def _enc_body(x_ref, w_ref, b_ref, q_ref, qn_ref):
    q = jax.lax.dot_general(
        x_ref[...].astype(jnp.bfloat16), w_ref[...].astype(jnp.bfloat16),
        (((1,), (0,)), ((), ())),
        preferred_element_type=jnp.float32)
    q = jnp.maximum(q + b_ref[...], 0.0)
    q_ref[...] = q
    norm = jnp.sqrt(jnp.sum(q * q, axis=1, keepdims=True))
    qn_ref[...] = q / jnp.maximum(norm, 1e-8)


def _encode(x, w, b):
    return pl.pallas_call(
        _enc_body,
        out_shape=(jax.ShapeDtypeStruct((Q, D), jnp.float32),
                   jax.ShapeDtypeStruct((Q, D), jnp.float32)),
    )(x, w, b.reshape(1, D))


def _sim_body(qn_ref, mem_ref, sim_ref):
    mem = mem_ref[...]
    rn = jnp.sqrt(jnp.sum(mem * mem, axis=1))
    rnorm = 1.0 / jnp.maximum(rn, 1e-8)
    mn = (mem * rnorm[:, None]).astype(jnp.bfloat16)
    s = jax.lax.dot_general(
        qn_ref[...].astype(jnp.bfloat16), mn, (((1,), (1,)), ((), ())),
        preferred_element_type=jnp.float32)
    j = pl.program_id(0)
    col = j * BN + jax.lax.broadcasted_iota(jnp.int32, s.shape, 1)
    sim_ref[...] = jnp.where(col < M, s, -jnp.inf)


def _similarity(qn, mem_pad):
    grid = (M_PAD // BN,)
    return pl.pallas_call(
        _sim_body,
        grid=grid,
        in_specs=[
            pl.BlockSpec((Q, D), lambda j: (0, 0)),
            pl.BlockSpec((BN, D), lambda j: (j, 0)),
        ],
        out_specs=pl.BlockSpec((Q, BN), lambda j: (0, j)),
        out_shape=jax.ShapeDtypeStruct((Q, M_PAD), jnp.float32),
    )(qn, mem_pad)


def kernel(query_input, memory_keys, W_enc, b_enc, Wq, bq, Wm, bm, Ws, bs,
           Wc, bc, k):
    del k
    q, qn = _encode(query_input, W_enc, b_enc)
    mem_pad = jnp.pad(memory_keys, ((0, M_PAD - M), (0, 0)))
    sim = _similarity(qn, mem_pad)
    _, idx = jax.lax.top_k(sim, K)
    knn = jnp.take(memory_keys, idx, axis=0)
    qa = (q @ Wq + bq)[:, None, :]
    ma = knn @ Wm + bm
    att = jnp.tanh(qa + ma)
    scores = att @ Ws + bs
    w = jax.nn.softmax(scores, axis=1)
    attended = jnp.sum(w * knn, axis=1)
    merged = jnp.concatenate([q, attended], axis=1)
    return merged @ Wc + bc


# speedup vs baseline: 3.7348x; 3.7348x over previous
import functools

import jax
import jax.numpy as jnp
from jax.experimental import pallas as pl
from jax.experimental.pallas import tpu as pltpu

Q, M, D, U, C, K = 1024, 100000, 512, 256, 1000, 64
BN = 2048
CH = 128
M_PAD = ((M + BN - 1) // BN) * BN
NCHUNK = M_PAD // CH
QB = 128
NEG = -3.0e38


def _enc_body(x_ref, w_ref, b_ref, q_ref, qn_ref):
    q = jax.lax.dot_general(
        x_ref[...].astype(jnp.bfloat16), w_ref[...].astype(jnp.bfloat16),
        (((1,), (0,)), ((), ())),
        preferred_element_type=jnp.float32)
    q = jnp.maximum(q + b_ref[...], 0.0)
    q_ref[...] = q
    norm = jnp.sqrt(jnp.sum(q * q, axis=1, keepdims=True))
    qn_ref[...] = q / jnp.maximum(norm, 1e-8)


def _encode(x, w, b):
    return pl.pallas_call(
        _enc_body,
        out_shape=(jax.ShapeDtypeStruct((Q, D), jnp.float32),
                   jax.ShapeDtypeStruct((Q, D), jnp.float32)),
    )(x, w, b.reshape(1, D))


def _sim_body(qn_ref, mem_ref, sim_ref, cmax_ref):
    mem = mem_ref[...]
    rn = jnp.sqrt(jnp.sum(mem * mem, axis=1))
    rnorm = 1.0 / jnp.maximum(rn, 1e-8)
    mn = (mem * rnorm[:, None]).astype(jnp.bfloat16)
    s = jax.lax.dot_general(
        qn_ref[...].astype(jnp.bfloat16), mn, (((1,), (1,)), ((), ())),
        preferred_element_type=jnp.float32)
    j = pl.program_id(0)
    col = j * BN + jax.lax.broadcasted_iota(jnp.int32, s.shape, 1)
    s = jnp.where(col < M, s, NEG)
    sim_ref[...] = s
    cmax_ref[...] = jnp.max(s.reshape(Q, BN // CH, CH), axis=2)[None]


def _similarity(qn, mem_pad):
    grid = (M_PAD // BN,)
    return pl.pallas_call(
        _sim_body,
        grid=grid,
        in_specs=[
            pl.BlockSpec((Q, D), lambda j: (0, 0)),
            pl.BlockSpec((BN, D), lambda j: (j, 0)),
        ],
        out_specs=(pl.BlockSpec((Q, BN), lambda j: (0, j)),
                   pl.BlockSpec((1, Q, BN // CH), lambda j: (j, 0, 0))),
        out_shape=(jax.ShapeDtypeStruct((Q, M_PAD), jnp.float32),
                   jax.ShapeDtypeStruct((M_PAD // BN, Q, BN // CH),
                                        jnp.float32)),
    )(qn, mem_pad)


def _topchunk_body(cmax_ref, ids_ref):
    cm = cmax_ref[...]
    jidx = jax.lax.broadcasted_iota(jnp.int32, cm.shape, 1)
    for t in range(K):
        m = jnp.max(cm, axis=1, keepdims=True)
        amin = jnp.min(jnp.where(cm >= m, jidx, NCHUNK), axis=1,
                       keepdims=True)
        ids_ref[:, t:t + 1] = amin
        cm = jnp.where(jidx == amin, NEG, cm)


def _top_chunks(cmax):
    return pl.pallas_call(
        _topchunk_body,
        out_shape=jax.ShapeDtypeStruct((Q, K), jnp.int32),
    )(cmax)


def _att_body(q_ref, knn_ref, wq_ref, bq_ref, wm_ref, bm_ref, ws_ref,
              bs_ref, wc_ref, bc_ref, out_ref):
    qb = q_ref[...]
    knn = knn_ref[...]
    qa = jax.lax.dot_general(
        qb, wq_ref[...], (((1,), (0,)), ((), ())),
        preferred_element_type=jnp.float32) + bq_ref[...]
    ma = jax.lax.dot_general(
        knn, wm_ref[...], (((1,), (0,)), ((), ())),
        preferred_element_type=jnp.float32) + bm_ref[...]
    att = jnp.tanh(qa.reshape(QB, 1, U) + ma.reshape(QB, K, U))
    sc = jax.lax.dot_general(
        att.reshape(QB * K, U), ws_ref[...], (((1,), (0,)), ((), ())),
        preferred_element_type=jnp.float32) + bs_ref[...]
    sc = sc.reshape(QB, K)
    sc = sc - jnp.max(sc, axis=1, keepdims=True)
    e = jnp.exp(sc)
    w = e / jnp.sum(e, axis=1, keepdims=True)
    attended = jnp.sum(w.reshape(QB, K, 1) * knn.reshape(QB, K, D), axis=1)
    merged = jnp.concatenate([qb, attended], axis=1)
    out_ref[...] = jax.lax.dot_general(
        merged, wc_ref[...], (((1,), (0,)), ((), ())),
        preferred_element_type=jnp.float32) + bc_ref[...]


def _attention(q, knn_flat, Wq, bq, Wm, bm, Ws, bs, Wc, bc):
    grid = (Q // QB,)
    return pl.pallas_call(
        _att_body,
        grid=grid,
        in_specs=[
            pl.BlockSpec((QB, D), lambda i: (i, 0)),
            pl.BlockSpec((QB * K, D), lambda i: (i, 0)),
            pl.BlockSpec((D, U), lambda i: (0, 0)),
            pl.BlockSpec((1, U), lambda i: (0, 0)),
            pl.BlockSpec((D, U), lambda i: (0, 0)),
            pl.BlockSpec((1, U), lambda i: (0, 0)),
            pl.BlockSpec((U, 1), lambda i: (0, 0)),
            pl.BlockSpec((1, 1), lambda i: (0, 0)),
            pl.BlockSpec((2 * D, C), lambda i: (0, 0)),
            pl.BlockSpec((1, C), lambda i: (0, 0)),
        ],
        out_specs=pl.BlockSpec((QB, C), lambda i: (i, 0)),
        out_shape=jax.ShapeDtypeStruct((Q, C), jnp.float32),
    )(q, knn_flat, Wq, bq.reshape(1, U), Wm, bm.reshape(1, U), Ws,
      bs.reshape(1, 1), Wc, bc.reshape(1, C))


def kernel(query_input, memory_keys, W_enc, b_enc, Wq, bq, Wm, bm, Ws, bs,
           Wc, bc, k):
    del k
    q, qn = _encode(query_input, W_enc, b_enc)
    mem_pad = jnp.pad(memory_keys, ((0, M_PAD - M), (0, 0)))
    sim, cmax3 = _similarity(qn, mem_pad)
    cmax = cmax3.transpose(1, 0, 2).reshape(Q, NCHUNK)
    chunk_ids = _top_chunks(cmax)
    cand = jnp.take_along_axis(
        sim.reshape(Q, NCHUNK, CH), chunk_ids[:, :, None], axis=1)
    cand = cand.reshape(Q, K * CH)
    _, loc = jax.lax.top_k(cand, K)
    idx = jnp.take_along_axis(chunk_ids, loc // CH, axis=1) * CH + loc % CH
    knn_flat = jnp.take(mem_pad, idx.reshape(-1), axis=0)
    return _attention(q, knn_flat, Wq, bq, Wm, bm, Ws, bs, Wc, bc)


# speedup vs baseline: 3.8412x; 1.0285x over previous
import functools

import jax
import jax.numpy as jnp
from jax.experimental import pallas as pl
from jax.experimental.pallas import tpu as pltpu

Q, M, D, U, C, K = 1024, 100000, 512, 256, 1000, 64
BN = 2048
CH = 128
M_PAD = ((M + BN - 1) // BN) * BN
NCHUNK = M_PAD // CH
QB = 128
NEG = -3.0e38


def _enc_body(x_ref, w_ref, b_ref, q_ref, qn_ref):
    q = jax.lax.dot_general(
        x_ref[...].astype(jnp.bfloat16), w_ref[...].astype(jnp.bfloat16),
        (((1,), (0,)), ((), ())),
        preferred_element_type=jnp.float32)
    q = jnp.maximum(q + b_ref[...], 0.0)
    q_ref[...] = q
    norm = jnp.sqrt(jnp.sum(q * q, axis=1, keepdims=True))
    qn_ref[...] = q / jnp.maximum(norm, 1e-8)


def _encode(x, w, b):
    return pl.pallas_call(
        _enc_body,
        out_shape=(jax.ShapeDtypeStruct((Q, D), jnp.float32),
                   jax.ShapeDtypeStruct((Q, D), jnp.float32)),
    )(x, w, b.reshape(1, D))


def _sim_body(qn_ref, mem_ref, sim_ref, cmax_ref):
    mem = mem_ref[...]
    rn = jnp.sqrt(jnp.sum(mem * mem, axis=1))
    rnorm = 1.0 / jnp.maximum(rn, 1e-8)
    mn = (mem * rnorm[:, None]).astype(jnp.bfloat16)
    s = jax.lax.dot_general(
        qn_ref[...].astype(jnp.bfloat16), mn, (((1,), (1,)), ((), ())),
        preferred_element_type=jnp.float32)
    j = pl.program_id(0)
    col = j * BN + jax.lax.broadcasted_iota(jnp.int32, s.shape, 1)
    s = jnp.where(col < M, s, NEG)
    sim_ref[...] = s
    cmax_ref[...] = jnp.max(s.reshape(Q, BN // CH, CH), axis=2)[None]


def _similarity(qn, mem):
    grid = (M_PAD // BN,)
    return pl.pallas_call(
        _sim_body,
        grid=grid,
        in_specs=[
            pl.BlockSpec((Q, D), lambda j: (0, 0)),
            pl.BlockSpec((BN, D), lambda j: (j, 0)),
        ],
        out_specs=(pl.BlockSpec((Q, BN), lambda j: (0, j)),
                   pl.BlockSpec((1, Q, BN // CH), lambda j: (j, 0, 0))),
        out_shape=(jax.ShapeDtypeStruct((Q, M_PAD), jnp.float32),
                   jax.ShapeDtypeStruct((M_PAD // BN, Q, BN // CH),
                                        jnp.float32)),
    )(qn, mem)


def _topchunk_body(cmax_ref, ids_ref):
    cm = cmax_ref[...]
    jidx = jax.lax.broadcasted_iota(jnp.int32, cm.shape, 1)
    for t in range(K):
        m = jnp.max(cm, axis=1, keepdims=True)
        amin = jnp.min(jnp.where(cm >= m, jidx, NCHUNK), axis=1,
                       keepdims=True)
        ids_ref[:, t:t + 1] = amin
        cm = jnp.where(jidx == amin, NEG, cm)


def _top_chunks(cmax):
    return pl.pallas_call(
        _topchunk_body,
        out_shape=jax.ShapeDtypeStruct((Q, K), jnp.int32),
    )(cmax)


def _att_body(q_ref, knn_ref, wq_ref, bq_ref, wm_ref, bm_ref, ws_ref,
              bs_ref, wc_ref, bc_ref, out_ref):
    qb = q_ref[...]
    knn = knn_ref[...]
    qa = jax.lax.dot_general(
        qb, wq_ref[...], (((1,), (0,)), ((), ())),
        preferred_element_type=jnp.float32) + bq_ref[...]
    ma = jax.lax.dot_general(
        knn, wm_ref[...], (((1,), (0,)), ((), ())),
        preferred_element_type=jnp.float32) + bm_ref[...]
    att = jnp.tanh(qa.reshape(QB, 1, U) + ma.reshape(QB, K, U))
    sc = jax.lax.dot_general(
        att.reshape(QB * K, U), ws_ref[...], (((1,), (0,)), ((), ())),
        preferred_element_type=jnp.float32) + bs_ref[...]
    sc = sc.reshape(QB, K)
    sc = sc - jnp.max(sc, axis=1, keepdims=True)
    e = jnp.exp(sc)
    w = e / jnp.sum(e, axis=1, keepdims=True)
    attended = jnp.sum(w.reshape(QB, K, 1) * knn.reshape(QB, K, D), axis=1)
    merged = jnp.concatenate([qb, attended], axis=1)
    out_ref[...] = jax.lax.dot_general(
        merged, wc_ref[...], (((1,), (0,)), ((), ())),
        preferred_element_type=jnp.float32) + bc_ref[...]


def _attention(q, knn_flat, Wq, bq, Wm, bm, Ws, bs, Wc, bc):
    grid = (Q // QB,)
    return pl.pallas_call(
        _att_body,
        grid=grid,
        in_specs=[
            pl.BlockSpec((QB, D), lambda i: (i, 0)),
            pl.BlockSpec((QB * K, D), lambda i: (i, 0)),
            pl.BlockSpec((D, U), lambda i: (0, 0)),
            pl.BlockSpec((1, U), lambda i: (0, 0)),
            pl.BlockSpec((D, U), lambda i: (0, 0)),
            pl.BlockSpec((1, U), lambda i: (0, 0)),
            pl.BlockSpec((U, 1), lambda i: (0, 0)),
            pl.BlockSpec((1, 1), lambda i: (0, 0)),
            pl.BlockSpec((2 * D, C), lambda i: (0, 0)),
            pl.BlockSpec((1, C), lambda i: (0, 0)),
        ],
        out_specs=pl.BlockSpec((QB, C), lambda i: (i, 0)),
        out_shape=jax.ShapeDtypeStruct((Q, C), jnp.float32),
    )(q, knn_flat, Wq, bq.reshape(1, U), Wm, bm.reshape(1, U), Ws,
      bs.reshape(1, 1), Wc, bc.reshape(1, C))


def kernel(query_input, memory_keys, W_enc, b_enc, Wq, bq, Wm, bm, Ws, bs,
           Wc, bc, k):
    del k
    q, qn = _encode(query_input, W_enc, b_enc)
    sim, cmax3 = _similarity(qn, memory_keys)
    cmax = cmax3.transpose(1, 0, 2).reshape(Q, NCHUNK)
    chunk_ids = _top_chunks(cmax)
    cand = jnp.take_along_axis(
        sim.reshape(Q, NCHUNK, CH), chunk_ids[:, :, None], axis=1)
    cand = cand.reshape(Q, K * CH)
    _, loc = jax.lax.top_k(cand, K)
    idx = jnp.take_along_axis(chunk_ids, loc // CH, axis=1) * CH + loc % CH
    knn_flat = jnp.take(memory_keys, idx.reshape(-1), axis=0)
    return _attention(q, knn_flat, Wq, bq, Wm, bm, Ws, bs, Wc, bc)


# speedup vs baseline: 13.0076x; 3.3864x over previous
import functools

import jax
import jax.numpy as jnp
from jax import lax
from jax.experimental import pallas as pl
from jax.experimental.pallas import tpu as pltpu
from jax.experimental.pallas import tpu_sc as plsc

Q, M, D, U, C, K = 1024, 100000, 512, 256, 1000, 64
BN = 2048
CH = 128
M_PAD = ((M + BN - 1) // BN) * BN
NCHUNK = M_PAD // CH
QB = 128
NEG = -3.0e38


def _enc_body(x_ref, w_ref, b_ref, q_ref, qn_ref):
    q = jax.lax.dot_general(
        x_ref[...].astype(jnp.bfloat16), w_ref[...].astype(jnp.bfloat16),
        (((1,), (0,)), ((), ())),
        preferred_element_type=jnp.float32)
    q = jnp.maximum(q + b_ref[...], 0.0)
    q_ref[...] = q
    norm = jnp.sqrt(jnp.sum(q * q, axis=1, keepdims=True))
    qn_ref[...] = q / jnp.maximum(norm, 1e-8)


def _encode(x, w, b):
    return pl.pallas_call(
        _enc_body,
        out_shape=(jax.ShapeDtypeStruct((Q, D), jnp.float32),
                   jax.ShapeDtypeStruct((Q, D), jnp.float32)),
    )(x, w, b.reshape(1, D))


def _sim_body(qn_ref, mem_ref, sim_ref, cmax_ref):
    mem = mem_ref[...]
    rn = jnp.sqrt(jnp.sum(mem * mem, axis=1))
    rnorm = 1.0 / jnp.maximum(rn, 1e-8)
    mn = (mem * rnorm[:, None]).astype(jnp.bfloat16)
    s = jax.lax.dot_general(
        qn_ref[...].astype(jnp.bfloat16), mn, (((1,), (1,)), ((), ())),
        preferred_element_type=jnp.float32)
    j = pl.program_id(0)
    col = j * BN + jax.lax.broadcasted_iota(jnp.int32, s.shape, 1)
    s = jnp.where(col < M, s, NEG)
    sim_ref[...] = s
    cmax_ref[...] = jnp.max(s.reshape(Q, BN // CH, CH), axis=2)[None]


def _similarity(qn, mem):
    grid = (M_PAD // BN,)
    return pl.pallas_call(
        _sim_body,
        grid=grid,
        in_specs=[
            pl.BlockSpec((Q, D), lambda j: (0, 0)),
            pl.BlockSpec((BN, D), lambda j: (j, 0)),
        ],
        out_specs=(pl.BlockSpec((Q, BN), lambda j: (0, j)),
                   pl.BlockSpec((1, Q, BN // CH), lambda j: (j, 0, 0))),
        out_shape=(jax.ShapeDtypeStruct((Q, M_PAD), jnp.float32),
                   jax.ShapeDtypeStruct((M_PAD // BN, Q, BN // CH),
                                        jnp.float32)),
    )(qn, mem)


def _topchunk_body(cmax_ref, ids_ref):
    cm = cmax_ref[...]
    jidx = jax.lax.broadcasted_iota(jnp.int32, cm.shape, 1)
    for t in range(K):
        m = jnp.max(cm, axis=1, keepdims=True)
        amin = jnp.min(jnp.where(cm >= m, jidx, NCHUNK), axis=1,
                       keepdims=True)
        ids_ref[:, t:t + 1] = amin
        cm = jnp.where(jidx == amin, NEG, cm)


def _top_chunks(cmax):
    return pl.pallas_call(
        _topchunk_body,
        out_shape=jax.ShapeDtypeStruct((Q, K), jnp.int32),
    )(cmax)


def _att_body(q_ref, knn_ref, wq_ref, bq_ref, wm_ref, bm_ref, ws_ref,
              bs_ref, wc_ref, bc_ref, out_ref):
    qb = q_ref[...]
    knn = knn_ref[...]
    qa = jax.lax.dot_general(
        qb, wq_ref[...], (((1,), (0,)), ((), ())),
        preferred_element_type=jnp.float32) + bq_ref[...]
    ma = jax.lax.dot_general(
        knn, wm_ref[...], (((1,), (0,)), ((), ())),
        preferred_element_type=jnp.float32) + bm_ref[...]
    att = jnp.tanh(qa.reshape(QB, 1, U) + ma.reshape(QB, K, U))
    sc = jax.lax.dot_general(
        att.reshape(QB * K, U), ws_ref[...], (((1,), (0,)), ((), ())),
        preferred_element_type=jnp.float32) + bs_ref[...]
    sc = sc.reshape(QB, K)
    sc = sc - jnp.max(sc, axis=1, keepdims=True)
    e = jnp.exp(sc)
    w = e / jnp.sum(e, axis=1, keepdims=True)
    attended = jnp.sum(w.reshape(QB, K, 1) * knn.reshape(QB, K, D), axis=1)
    merged = jnp.concatenate([qb, attended], axis=1)
    out_ref[...] = jax.lax.dot_general(
        merged, wc_ref[...], (((1,), (0,)), ((), ())),
        preferred_element_type=jnp.float32) + bc_ref[...]


def _attention(q, knn_flat, Wq, bq, Wm, bm, Ws, bs, Wc, bc):
    grid = (Q // QB,)
    return pl.pallas_call(
        _att_body,
        grid=grid,
        in_specs=[
            pl.BlockSpec((QB, D), lambda i: (i, 0)),
            pl.BlockSpec((QB * K, D), lambda i: (i, 0)),
            pl.BlockSpec((D, U), lambda i: (0, 0)),
            pl.BlockSpec((1, U), lambda i: (0, 0)),
            pl.BlockSpec((D, U), lambda i: (0, 0)),
            pl.BlockSpec((1, U), lambda i: (0, 0)),
            pl.BlockSpec((U, 1), lambda i: (0, 0)),
            pl.BlockSpec((1, 1), lambda i: (0, 0)),
            pl.BlockSpec((2 * D, C), lambda i: (0, 0)),
            pl.BlockSpec((1, C), lambda i: (0, 0)),
        ],
        out_specs=pl.BlockSpec((QB, C), lambda i: (i, 0)),
        out_shape=jax.ShapeDtypeStruct((Q, C), jnp.float32),
    )(q, knn_flat, Wq, bq.reshape(1, U), Wm, bm.reshape(1, U), Ws,
      bs.reshape(1, 1), Wc, bc.reshape(1, C))


NC, NS, L = 2, 16, 16
NW = NC * NS
QPW = Q // NW


def _sc_sel_body(simtab_hbm, cid_hbm, out_hbm, cid_v, gidx_v, rows_v,
                 keep_v, keepi_v, outi_v, sem):
    wid = lax.axis_index("s") * NC + lax.axis_index("c")
    lanes = lax.broadcasted_iota(jnp.int32, (L,), 0)

    def smax(v):
        sk, _ = plsc.sort_key_val(v, lanes, descending=True)
        return sk[0]

    def lane_get(v, j):
        idxs = ((lanes + j) % L).reshape(L, 1)
        return lax.gather(
            v, idxs,
            lax.GatherDimensionNumbers(offset_dims=(),
                                       collapsed_slice_dims=(0,),
                                       start_index_map=(0,)),
            (1,), mode=lax.GatherScatterMode.PROMISE_IN_BOUNDS)[0]

    def per_query(t, carry):
        q = wid * QPW + t
        pltpu.sync_copy(cid_hbm.at[q], cid_v)
        for i in range(K // L):
            gidx_v[pl.ds(i * L, L)] = cid_v[pl.ds(i * L, L)] + q * NCHUNK
        pltpu.async_copy(simtab_hbm.at[gidx_v], rows_v, sem).wait()

        def chunk_max(c, thr):
            m = rows_v[c, pl.ds(0, L)]
            for i in range(1, CH // L):
                m = jnp.maximum(m, rows_v[c, pl.ds(i * L, L)])
            return jnp.minimum(thr, smax(m))

        thr = lax.fori_loop(0, K, chunk_max, jnp.float32(3.0e38))

        def compact(c, off):
            grp = (c // L) * L
            cvec = cid_v[pl.ds(grp, L)]
            base = lane_get(cvec, c % L) * CH
            for i in range(CH // L):
                v = rows_v[c, pl.ds(i * L, L)]
                msk = v >= thr
                cnt = plsc.all_reduce_population_count(msk)[0]
                gi = base + i * L + lanes
                plsc.store_compressed(keep_v.at[pl.ds(off, L)], v, mask=msk)
                plsc.store_compressed(keepi_v.at[pl.ds(off, L)], gi,
                                      mask=msk)
                off = off + cnt
            return off

        n = lax.fori_loop(0, K, compact, jnp.int32(0))
        keep_v[pl.ds(n, L)] = jnp.full((L,), NEG, jnp.float32)
        nv = (n + L - 1) // L

        def select(i, carry):
            def scan(j, bc):
                b, bj = bc
                m = smax(keep_v[pl.ds(j * L, L)])
                better = m > b
                return (jnp.where(better, m, b), jnp.where(better, j, bj))

            best, bestj = lax.fori_loop(0, nv, scan, (jnp.float32(NEG),
                                                      jnp.int32(0)))
            v = keep_v[pl.ds(bestj * L, L)]
            fm = v == best
            fl = plsc.all_reduce_ffs(fm)[0]
            fm = lanes == fl
            gi = lane_get(keepi_v[pl.ds(bestj * L, L)], fl)
            og = (i // L) * L
            ovec = outi_v[pl.ds(og, L)]
            outi_v[pl.ds(og, L)] = jnp.where(lanes == i % L, gi, ovec)
            keep_v[pl.ds(bestj * L, L)] = jnp.where(fm, NEG, v)
            return carry

        lax.fori_loop(0, K, select, 0)
        pltpu.sync_copy(outi_v, out_hbm.at[q])
        return carry

    lax.fori_loop(0, QPW, per_query, 0)


def _sc_select(sim, chunk_ids):
    simtab = sim.reshape(Q * NCHUNK, CH)
    mesh = plsc.VectorSubcoreMesh(core_axis_name="c", subcore_axis_name="s",
                                  num_cores=NC, num_subcores=NS)
    return pl.kernel(
        _sc_sel_body,
        out_type=jax.ShapeDtypeStruct((Q, K), jnp.int32),
        mesh=mesh,
        compiler_params=pltpu.CompilerParams(needs_layout_passes=False),
        scratch_types=[
            pltpu.VMEM((K,), jnp.int32),
            pltpu.VMEM((K,), jnp.int32),
            pltpu.VMEM((K, CH), jnp.float32),
            pltpu.VMEM((K * CH + L,), jnp.float32),
            pltpu.VMEM((K * CH,), jnp.int32),
            pltpu.VMEM((K,), jnp.int32),
            pltpu.SemaphoreType.DMA,
        ],
    )(simtab, chunk_ids)


def kernel(query_input, memory_keys, W_enc, b_enc, Wq, bq, Wm, bm, Ws, bs,
           Wc, bc, k):
    del k
    q, qn = _encode(query_input, W_enc, b_enc)
    sim, cmax3 = _similarity(qn, memory_keys)
    cmax = cmax3.transpose(1, 0, 2).reshape(Q, NCHUNK)
    chunk_ids = _top_chunks(cmax)
    idx = _sc_select(sim, chunk_ids)
    knn_flat = jnp.take(memory_keys, idx.reshape(-1), axis=0)
    return _attention(q, knn_flat, Wq, bq, Wm, bm, Ws, bs, Wc, bc)


# speedup vs baseline: 13.8828x; 1.0673x over previous
import functools

import jax
import jax.numpy as jnp
from jax import lax
from jax.experimental import pallas as pl
from jax.experimental.pallas import tpu as pltpu
from jax.experimental.pallas import tpu_sc as plsc

Q, M, D, U, C, K = 1024, 100000, 512, 256, 1000, 64
BN = 2048
CH = 128
M_PAD = ((M + BN - 1) // BN) * BN
NCHUNK = M_PAD // CH
QB = 128
NEG = -3.0e38


def _enc_body(x_ref, w_ref, b_ref, q_ref, qn_ref):
    q = jax.lax.dot_general(
        x_ref[...].astype(jnp.bfloat16), w_ref[...].astype(jnp.bfloat16),
        (((1,), (0,)), ((), ())),
        preferred_element_type=jnp.float32)
    q = jnp.maximum(q + b_ref[...], 0.0)
    q_ref[...] = q
    norm = jnp.sqrt(jnp.sum(q * q, axis=1, keepdims=True))
    qn_ref[...] = q / jnp.maximum(norm, 1e-8)


def _encode(x, w, b):
    return pl.pallas_call(
        _enc_body,
        out_shape=(jax.ShapeDtypeStruct((Q, D), jnp.float32),
                   jax.ShapeDtypeStruct((Q, D), jnp.float32)),
    )(x, w, b.reshape(1, D))


def _sim_body(qn_ref, mem_ref, sim_ref, cmax_ref):
    mem = mem_ref[...]
    rn = jnp.sqrt(jnp.sum(mem * mem, axis=1))
    rnorm = 1.0 / jnp.maximum(rn, 1e-8)
    mn = (mem * rnorm[:, None]).astype(jnp.bfloat16)
    s = jax.lax.dot_general(
        qn_ref[...].astype(jnp.bfloat16), mn, (((1,), (1,)), ((), ())),
        preferred_element_type=jnp.float32)
    j = pl.program_id(0)
    col = j * BN + jax.lax.broadcasted_iota(jnp.int32, s.shape, 1)
    s = jnp.where(col < M, s, NEG)
    sim_ref[...] = s
    cmax_ref[...] = jnp.max(s.reshape(Q, BN // CH, CH), axis=2)[None]


def _similarity(qn, mem):
    grid = (M_PAD // BN,)
    return pl.pallas_call(
        _sim_body,
        grid=grid,
        in_specs=[
            pl.BlockSpec((Q, D), lambda j: (0, 0)),
            pl.BlockSpec((BN, D), lambda j: (j, 0)),
        ],
        out_specs=(pl.BlockSpec((Q, BN), lambda j: (0, j)),
                   pl.BlockSpec((1, Q, BN // CH), lambda j: (j, 0, 0))),
        out_shape=(jax.ShapeDtypeStruct((Q, M_PAD), jnp.float32),
                   jax.ShapeDtypeStruct((M_PAD // BN, Q, BN // CH),
                                        jnp.float32)),
    )(qn, mem)


def _topchunk_body(cmax_ref, ids_ref):
    cm = cmax_ref[...]
    jidx = jax.lax.broadcasted_iota(jnp.int32, cm.shape, 1)
    for t in range(K):
        m = jnp.max(cm, axis=1, keepdims=True)
        amin = jnp.min(jnp.where(cm >= m, jidx, NCHUNK), axis=1,
                       keepdims=True)
        ids_ref[:, t:t + 1] = amin
        cm = jnp.where(jidx == amin, NEG, cm)


def _top_chunks(cmax):
    return pl.pallas_call(
        _topchunk_body,
        out_shape=jax.ShapeDtypeStruct((Q, K), jnp.int32),
    )(cmax)


def _att_body(q_ref, knn_ref, wq_ref, bq_ref, wm_ref, bm_ref, ws_ref,
              bs_ref, wc_ref, bc_ref, out_ref):
    qb = q_ref[...]
    knn = knn_ref[...]
    qa = jax.lax.dot_general(
        qb, wq_ref[...], (((1,), (0,)), ((), ())),
        preferred_element_type=jnp.float32) + bq_ref[...]
    ma = jax.lax.dot_general(
        knn, wm_ref[...], (((1,), (0,)), ((), ())),
        preferred_element_type=jnp.float32) + bm_ref[...]
    att = jnp.tanh(qa.reshape(QB, 1, U) + ma.reshape(QB, K, U))
    sc = jax.lax.dot_general(
        att.reshape(QB * K, U), ws_ref[...], (((1,), (0,)), ((), ())),
        preferred_element_type=jnp.float32) + bs_ref[...]
    sc = sc.reshape(QB, K)
    sc = sc - jnp.max(sc, axis=1, keepdims=True)
    e = jnp.exp(sc)
    w = e / jnp.sum(e, axis=1, keepdims=True)
    attended = jnp.sum(w.reshape(QB, K, 1) * knn.reshape(QB, K, D), axis=1)
    merged = jnp.concatenate([qb, attended], axis=1)
    out_ref[...] = jax.lax.dot_general(
        merged, wc_ref[...], (((1,), (0,)), ((), ())),
        preferred_element_type=jnp.float32) + bc_ref[...]


def _attention(q, knn_flat, Wq, bq, Wm, bm, Ws, bs, Wc, bc):
    grid = (Q // QB,)
    return pl.pallas_call(
        _att_body,
        grid=grid,
        in_specs=[
            pl.BlockSpec((QB, D), lambda i: (i, 0)),
            pl.BlockSpec((QB * K, D), lambda i: (i, 0)),
            pl.BlockSpec((D, U), lambda i: (0, 0)),
            pl.BlockSpec((1, U), lambda i: (0, 0)),
            pl.BlockSpec((D, U), lambda i: (0, 0)),
            pl.BlockSpec((1, U), lambda i: (0, 0)),
            pl.BlockSpec((U, 1), lambda i: (0, 0)),
            pl.BlockSpec((1, 1), lambda i: (0, 0)),
            pl.BlockSpec((2 * D, C), lambda i: (0, 0)),
            pl.BlockSpec((1, C), lambda i: (0, 0)),
        ],
        out_specs=pl.BlockSpec((QB, C), lambda i: (i, 0)),
        out_shape=jax.ShapeDtypeStruct((Q, C), jnp.float32),
    )(q, knn_flat, Wq, bq.reshape(1, U), Wm, bm.reshape(1, U), Ws,
      bs.reshape(1, 1), Wc, bc.reshape(1, C))


NC, NS, L = 2, 16, 16
NW = NC * NS
QPW = Q // NW


def _sc_sel_body(simtab_hbm, cid_hbm, mem_hbm, knn_hbm, cid_v, gidx_v,
                 rows_v, keep_v, keepi_v, outi_v, knnrow_v, sem):
    wid = lax.axis_index("s") * NC + lax.axis_index("c")
    lanes = lax.broadcasted_iota(jnp.int32, (L,), 0)

    def smax(v):
        sk, _ = plsc.sort_key_val(v, lanes, descending=True)
        return sk[0]

    def lane_get(v, j):
        idxs = ((lanes + j) % L).reshape(L, 1)
        return lax.gather(
            v, idxs,
            lax.GatherDimensionNumbers(offset_dims=(),
                                       collapsed_slice_dims=(0,),
                                       start_index_map=(0,)),
            (1,), mode=lax.GatherScatterMode.PROMISE_IN_BOUNDS)[0]

    def per_query(t, carry):
        q = wid * QPW + t
        pltpu.sync_copy(cid_hbm.at[q], cid_v)
        for i in range(K // L):
            gidx_v[pl.ds(i * L, L)] = cid_v[pl.ds(i * L, L)] + q * NCHUNK
        pltpu.async_copy(simtab_hbm.at[gidx_v], rows_v, sem).wait()

        def chunk_max(c, thr):
            m = rows_v[c, pl.ds(0, L)]
            for i in range(1, CH // L):
                m = jnp.maximum(m, rows_v[c, pl.ds(i * L, L)])
            return jnp.minimum(thr, smax(m))

        thr = lax.fori_loop(0, K, chunk_max, jnp.float32(3.0e38))

        def compact(c, off):
            grp = (c // L) * L
            cvec = cid_v[pl.ds(grp, L)]
            base = lane_get(cvec, c % L) * CH
            for i in range(CH // L):
                v = rows_v[c, pl.ds(i * L, L)]
                msk = v >= thr
                cnt = plsc.all_reduce_population_count(msk)[0]
                gi = base + i * L + lanes
                plsc.store_compressed(keep_v.at[pl.ds(off, L)], v, mask=msk)
                plsc.store_compressed(keepi_v.at[pl.ds(off, L)], gi,
                                      mask=msk)
                off = off + cnt
            return off

        n = lax.fori_loop(0, K, compact, jnp.int32(0))
        keep_v[pl.ds(n, L)] = jnp.full((L,), NEG, jnp.float32)
        nv = (n + L - 1) // L

        def select(i, carry):
            def scan(j, bc):
                b, bj = bc
                m = smax(keep_v[pl.ds(j * L, L)])
                better = m > b
                return (jnp.where(better, m, b), jnp.where(better, j, bj))

            best, bestj = lax.fori_loop(0, nv, scan, (jnp.float32(NEG),
                                                      jnp.int32(0)))
            v = keep_v[pl.ds(bestj * L, L)]
            fm = v == best
            fl = plsc.all_reduce_ffs(fm)[0]
            fm = lanes == fl
            gi = lane_get(keepi_v[pl.ds(bestj * L, L)], fl)
            og = (i // L) * L
            ovec = outi_v[pl.ds(og, L)]
            outi_v[pl.ds(og, L)] = jnp.where(lanes == i % L, gi, ovec)
            keep_v[pl.ds(bestj * L, L)] = jnp.where(fm, NEG, v)
            return carry

        lax.fori_loop(0, K, select, 0)
        pltpu.async_copy(mem_hbm.at[outi_v], knnrow_v, sem).wait()
        pltpu.sync_copy(knnrow_v, knn_hbm.at[pl.ds(q * K, K)])
        return carry

    lax.fori_loop(0, QPW, per_query, 0)


def _sc_select(sim, chunk_ids, mem):
    simtab = sim.reshape(Q * NCHUNK, CH)
    mesh = plsc.VectorSubcoreMesh(core_axis_name="c", subcore_axis_name="s",
                                  num_cores=NC, num_subcores=NS)
    return pl.kernel(
        _sc_sel_body,
        out_type=jax.ShapeDtypeStruct((Q * K, D), jnp.float32),
        mesh=mesh,
        compiler_params=pltpu.CompilerParams(needs_layout_passes=False),
        scratch_types=[
            pltpu.VMEM((K,), jnp.int32),
            pltpu.VMEM((K,), jnp.int32),
            pltpu.VMEM((K, CH), jnp.float32),
            pltpu.VMEM((K * CH + L,), jnp.float32),
            pltpu.VMEM((K * CH,), jnp.int32),
            pltpu.VMEM((K,), jnp.int32),
            pltpu.VMEM((K, D), jnp.float32),
            pltpu.SemaphoreType.DMA,
        ],
    )(simtab, chunk_ids, mem)


def kernel(query_input, memory_keys, W_enc, b_enc, Wq, bq, Wm, bm, Ws, bs,
           Wc, bc, k):
    del k
    q, qn = _encode(query_input, W_enc, b_enc)
    sim, cmax3 = _similarity(qn, memory_keys)
    cmax = cmax3.transpose(1, 0, 2).reshape(Q, NCHUNK)
    chunk_ids = _top_chunks(cmax)
    knn_flat = _sc_select(sim, chunk_ids, memory_keys)
    return _attention(q, knn_flat, Wq, bq, Wm, bm, Ws, bs, Wc, bc)


# speedup vs baseline: 13.8834x; 1.0000x over previous
import jax
import jax.numpy as jnp
from jax import lax
from jax.experimental import pallas as pl
from jax.experimental.pallas import tpu as pltpu
from jax.experimental.pallas import tpu_sc as plsc

Q, M, D, U, C, K = 1024, 100000, 512, 256, 1000, 64
BN = 2048
CH = 128
M_PAD = ((M + BN - 1) // BN) * BN
NCHUNK = M_PAD // CH
QB = 128
NEG = -3.0e38


def _enc_body(x_ref, w_ref, b_ref, q_ref, qn_ref):
    q = jax.lax.dot_general(
        x_ref[...].astype(jnp.bfloat16), w_ref[...].astype(jnp.bfloat16),
        (((1,), (0,)), ((), ())),
        preferred_element_type=jnp.float32)
    q = jnp.maximum(q + b_ref[...], 0.0)
    q_ref[...] = q
    norm = jnp.sqrt(jnp.sum(q * q, axis=1, keepdims=True))
    qn_ref[...] = q / jnp.maximum(norm, 1e-8)


def _encode(x, w, b):
    return pl.pallas_call(
        _enc_body,
        out_shape=(jax.ShapeDtypeStruct((Q, D), jnp.float32),
                   jax.ShapeDtypeStruct((Q, D), jnp.float32)),
    )(x, w, b.reshape(1, D))


def _sim_body(qn_ref, mem_ref, sim_ref, cmax_ref):
    mem = mem_ref[...]
    rn = jnp.sqrt(jnp.sum(mem * mem, axis=1))
    rnorm = 1.0 / jnp.maximum(rn, 1e-8)
    mn = (mem * rnorm[:, None]).astype(jnp.bfloat16)
    s = jax.lax.dot_general(
        qn_ref[...].astype(jnp.bfloat16), mn, (((1,), (1,)), ((), ())),
        preferred_element_type=jnp.float32)
    j = pl.program_id(0)
    col = j * BN + jax.lax.broadcasted_iota(jnp.int32, s.shape, 1)
    s = jnp.where(col < M, s, NEG)
    sim_ref[...] = s
    cmax_ref[...] = jnp.max(s.reshape(Q, BN // CH, CH), axis=2)[None]


def _similarity(qn, mem):
    grid = (M_PAD // BN,)
    return pl.pallas_call(
        _sim_body,
        grid=grid,
        in_specs=[
            pl.BlockSpec((Q, D), lambda j: (0, 0)),
            pl.BlockSpec((BN, D), lambda j: (j, 0)),
        ],
        out_specs=(pl.BlockSpec((Q, BN), lambda j: (0, j)),
                   pl.BlockSpec((1, Q, BN // CH), lambda j: (j, 0, 0))),
        out_shape=(jax.ShapeDtypeStruct((Q, M_PAD), jnp.float32),
                   jax.ShapeDtypeStruct((M_PAD // BN, Q, BN // CH),
                                        jnp.float32)),
    )(qn, mem)


def _topchunk_body(cmax_ref, ids_ref):
    cm = cmax_ref[...]
    jidx = jax.lax.broadcasted_iota(jnp.int32, cm.shape, 1)
    for t in range(K):
        m = jnp.max(cm, axis=1, keepdims=True)
        amin = jnp.min(jnp.where(cm >= m, jidx, NCHUNK), axis=1,
                       keepdims=True)
        ids_ref[:, t:t + 1] = amin
        cm = jnp.where(jidx == amin, NEG, cm)


def _top_chunks(cmax):
    return pl.pallas_call(
        _topchunk_body,
        out_shape=jax.ShapeDtypeStruct((Q, K), jnp.int32),
    )(cmax)


def _att_body(q_ref, knn_ref, wq_ref, bq_ref, wm_ref, bm_ref, ws_ref,
              bs_ref, wc_ref, bc_ref, out_ref):
    qb = q_ref[...]
    knn = knn_ref[...]
    qa = jax.lax.dot_general(
        qb, wq_ref[...], (((1,), (0,)), ((), ())),
        preferred_element_type=jnp.float32) + bq_ref[...]
    ma = jax.lax.dot_general(
        knn, wm_ref[...], (((1,), (0,)), ((), ())),
        preferred_element_type=jnp.float32) + bm_ref[...]
    att = jnp.tanh(qa.reshape(QB, 1, U) + ma.reshape(QB, K, U))
    sc = jax.lax.dot_general(
        att.reshape(QB * K, U), ws_ref[...], (((1,), (0,)), ((), ())),
        preferred_element_type=jnp.float32) + bs_ref[...]
    sc = sc.reshape(QB, K)
    sc = sc - jnp.max(sc, axis=1, keepdims=True)
    e = jnp.exp(sc)
    w = e / jnp.sum(e, axis=1, keepdims=True)
    attended = jnp.sum(w.reshape(QB, K, 1) * knn.reshape(QB, K, D), axis=1)
    merged = jnp.concatenate([qb, attended], axis=1)
    out_ref[...] = jax.lax.dot_general(
        merged, wc_ref[...], (((1,), (0,)), ((), ())),
        preferred_element_type=jnp.float32) + bc_ref[...]


def _attention(q, knn_flat, Wq, bq, Wm, bm, Ws, bs, Wc, bc):
    grid = (Q // QB,)
    return pl.pallas_call(
        _att_body,
        grid=grid,
        in_specs=[
            pl.BlockSpec((QB, D), lambda i: (i, 0)),
            pl.BlockSpec((QB * K, D), lambda i: (i, 0)),
            pl.BlockSpec((D, U), lambda i: (0, 0)),
            pl.BlockSpec((1, U), lambda i: (0, 0)),
            pl.BlockSpec((D, U), lambda i: (0, 0)),
            pl.BlockSpec((1, U), lambda i: (0, 0)),
            pl.BlockSpec((U, 1), lambda i: (0, 0)),
            pl.BlockSpec((1, 1), lambda i: (0, 0)),
            pl.BlockSpec((2 * D, C), lambda i: (0, 0)),
            pl.BlockSpec((1, C), lambda i: (0, 0)),
        ],
        out_specs=pl.BlockSpec((QB, C), lambda i: (i, 0)),
        out_shape=jax.ShapeDtypeStruct((Q, C), jnp.float32),
    )(q, knn_flat, Wq, bq.reshape(1, U), Wm, bm.reshape(1, U), Ws,
      bs.reshape(1, 1), Wc, bc.reshape(1, C))


NC, NS, L = 2, 16, 16
NW = NC * NS
QPW = Q // NW


def _sc_sel_body(simtab_hbm, cid_hbm, mem_hbm, knn_hbm, cid_v, gidx_v,
                 rows_v, keep_v, keepi_v, outi_v, knnrow_v, sem):
    wid = lax.axis_index("s") * NC + lax.axis_index("c")
    lanes = lax.broadcasted_iota(jnp.int32, (L,), 0)

    def smax(v):
        sk, _ = plsc.sort_key_val(v, lanes, descending=True)
        return sk[0]

    def lane_get(v, j):
        idxs = ((lanes + j) % L).reshape(L, 1)
        return lax.gather(
            v, idxs,
            lax.GatherDimensionNumbers(offset_dims=(),
                                       collapsed_slice_dims=(0,),
                                       start_index_map=(0,)),
            (1,), mode=lax.GatherScatterMode.PROMISE_IN_BOUNDS)[0]

    def per_query(t, carry):
        q = wid * QPW + t
        pltpu.sync_copy(cid_hbm.at[q], cid_v)
        for i in range(K // L):
            gidx_v[pl.ds(i * L, L)] = cid_v[pl.ds(i * L, L)] + q * NCHUNK
        pltpu.async_copy(simtab_hbm.at[gidx_v], rows_v, sem).wait()

        def chunk_max(c, thr):
            m = rows_v[c, pl.ds(0, L)]
            for i in range(1, CH // L):
                m = jnp.maximum(m, rows_v[c, pl.ds(i * L, L)])
            return jnp.minimum(thr, smax(m))

        thr = lax.fori_loop(0, K, chunk_max, jnp.float32(3.0e38))

        def compact(c, off):
            grp = (c // L) * L
            cvec = cid_v[pl.ds(grp, L)]
            base = lane_get(cvec, c % L) * CH
            for i in range(CH // L):
                v = rows_v[c, pl.ds(i * L, L)]
                msk = v >= thr
                cnt = plsc.all_reduce_population_count(msk)[0]
                gi = base + i * L + lanes
                plsc.store_compressed(keep_v.at[pl.ds(off, L)], v, mask=msk)
                plsc.store_compressed(keepi_v.at[pl.ds(off, L)], gi,
                                      mask=msk)
                off = off + cnt
            return off

        n = lax.fori_loop(0, K, compact, jnp.int32(0))
        keep_v[pl.ds(n, L)] = jnp.full((L,), NEG, jnp.float32)
        nv = (n + L - 1) // L

        def select(i, carry):
            def scan(j, bc):
                b, bj = bc
                m = smax(keep_v[pl.ds(j * L, L)])
                better = m > b
                return (jnp.where(better, m, b), jnp.where(better, j, bj))

            best, bestj = lax.fori_loop(0, nv, scan, (jnp.float32(NEG),
                                                      jnp.int32(0)))
            v = keep_v[pl.ds(bestj * L, L)]
            fm = v == best
            fl = plsc.all_reduce_ffs(fm)[0]
            fm = lanes == fl
            gi = lane_get(keepi_v[pl.ds(bestj * L, L)], fl)
            og = (i // L) * L
            ovec = outi_v[pl.ds(og, L)]
            outi_v[pl.ds(og, L)] = jnp.where(lanes == i % L, gi, ovec)
            keep_v[pl.ds(bestj * L, L)] = jnp.where(fm, NEG, v)
            return carry

        lax.fori_loop(0, K, select, 0)
        pltpu.async_copy(mem_hbm.at[outi_v], knnrow_v, sem).wait()
        pltpu.sync_copy(knnrow_v, knn_hbm.at[pl.ds(q * K, K)])
        return carry

    lax.fori_loop(0, QPW, per_query, 0)


def _sc_select(sim, chunk_ids, mem):
    simtab = sim.reshape(Q * NCHUNK, CH)
    mesh = plsc.VectorSubcoreMesh(core_axis_name="c", subcore_axis_name="s",
                                  num_cores=NC, num_subcores=NS)
    return pl.kernel(
        _sc_sel_body,
        out_type=jax.ShapeDtypeStruct((Q * K, D), jnp.float32),
        mesh=mesh,
        compiler_params=pltpu.CompilerParams(needs_layout_passes=False),
        scratch_types=[
            pltpu.VMEM((K,), jnp.int32),
            pltpu.VMEM((K,), jnp.int32),
            pltpu.VMEM((K, CH), jnp.float32),
            pltpu.VMEM((K * CH + L,), jnp.float32),
            pltpu.VMEM((K * CH,), jnp.int32),
            pltpu.VMEM((K,), jnp.int32),
            pltpu.VMEM((K, D), jnp.float32),
            pltpu.SemaphoreType.DMA,
        ],
    )(simtab, chunk_ids, mem)


def kernel(query_input, memory_keys, W_enc, b_enc, Wq, bq, Wm, bm, Ws, bs,
           Wc, bc, k):
    del k
    q, qn = _encode(query_input, W_enc, b_enc)
    sim, cmax3 = _similarity(qn, memory_keys)
    cmax = cmax3.transpose(1, 0, 2).reshape(Q, NCHUNK)
    chunk_ids = _top_chunks(cmax)
    knn_flat = _sc_select(sim, chunk_ids, memory_keys)
    return _attention(q, knn_flat, Wq, bq, Wm, bm, Ws, bs, Wc, bc)
